# baseline (speedup 1.0000x reference)
import functools

import jax
import jax.numpy as jnp
from jax.experimental import pallas as pl
from jax.experimental.pallas import tpu as pltpu

B, SL, DL, D = 1024, 200, 200, 64
L2 = SL + DL
RB = 8


def _body(src_ref, dst_ref, w1_ref, b1_ref, w2_ref, b2_ref, src_out, dst_out):
    src = src_ref[...]
    dst = dst_ref[...]
    ids = jnp.concatenate([src, dst], axis=1)
    idsf = ids.astype(jnp.float32)
    eq = jnp.where((idsf[:, :, None] == idsf[:, None, :])
                   & (idsf[:, :, None] != 0.0), 1.0, 0.0)
    c0 = eq[:, :, :SL].sum(-1)
    c1 = eq[:, :, SL:].sum(-1)
    w1 = w1_ref[0, :]
    b1 = b1_ref[...]
    h = (jnp.maximum(c0[:, :, None] * w1[None, None, :] + b1[None], 0.0)
         + jnp.maximum(c1[:, :, None] * w1[None, None, :] + b1[None], 0.0))
    feat = jnp.dot(h.reshape(RB * L2, D), w2_ref[...],
                   preferred_element_type=jnp.float32)
    feat = feat + 2.0 * b2_ref[...]
    feat = feat.reshape(RB, L2, D)
    src_out[...] = feat[:, :SL, :]
    dst_out[...] = feat[:, SL:, :]


@jax.jit
def kernel(src_ids, dst_ids, W1, b1, W2, b2):
    grid = (B // RB,)
    src_feat, dst_feat = pl.pallas_call(
        _body,
        grid=grid,
        in_specs=[
            pl.BlockSpec((RB, SL), lambda i: (i, 0)),
            pl.BlockSpec((RB, DL), lambda i: (i, 0)),
            pl.BlockSpec((1, D), lambda i: (0, 0)),
            pl.BlockSpec((1, D), lambda i: (0, 0)),
            pl.BlockSpec((D, D), lambda i: (0, 0)),
            pl.BlockSpec((1, D), lambda i: (0, 0)),
        ],
        out_specs=[
            pl.BlockSpec((RB, SL, D), lambda i: (i, 0, 0)),
            pl.BlockSpec((RB, DL, D), lambda i: (i, 0, 0)),
        ],
        out_shape=[
            jax.ShapeDtypeStruct((B, SL, D), jnp.float32),
            jax.ShapeDtypeStruct((B, DL, D), jnp.float32),
        ],
    )(src_ids, dst_ids, W1, b1.reshape(1, D), W2, b2.reshape(1, D))
    return src_feat, dst_feat


# speedup vs baseline: 1.2230x; 1.2230x over previous
import functools

import jax
import jax.numpy as jnp
from jax import lax
from jax.experimental import pallas as pl
from jax.experimental.pallas import tpu as pltpu
from jax.experimental.pallas import tpu_sc as plsc

B, SL, DL, D = 1024, 200, 200, 64
L2 = SL + DL
LP = 512
NC, NS = 2, 16
NW = NC * NS
ROWS_PER_W = B // NW
HSIZE = 100000
RB = 8

_mesh = plsc.VectorSubcoreMesh(core_axis_name="c", subcore_axis_name="s",
                               num_cores=NC, num_subcores=NS)


def _sc_count_body(ids_hbm, wvec_hbm, zvec_hbm, c0_hbm, c1_hbm,
                   hist, ids_v, wvec_v, zvec_v, idx_v, cnt_v, c0_v, c1_v):
    c = lax.axis_index("c")
    s = lax.axis_index("s")
    wid = c * NS + s
    pltpu.sync_copy(wvec_hbm, wvec_v)
    pltpu.sync_copy(zvec_hbm, zvec_v)

    def row_step(r, carry):
        row = wid * ROWS_PER_W + r
        pltpu.sync_copy(ids_hbm.at[row], ids_v)
        for j in range(4):
            for k in range(8):
                sl = pl.ds(k * 16, 16)
                idx_v[j, sl] = ids_v[j, sl] + s * HSIZE
        for j in range(4):
            pltpu.sync_copy(zvec_v.at[j], hist.at[idx_v.at[j]])
        for j in range(4):
            pltpu.sync_copy(wvec_v.at[j], hist.at[idx_v.at[j]], add=True)
        for j in range(4):
            pltpu.sync_copy(hist.at[idx_v.at[j]], cnt_v.at[j])
        for j in range(4):
            for k in range(8):
                sl = pl.ds(k * 16, 16)
                ids16 = ids_v[j, sl]
                cnt16 = cnt_v[j, sl]
                nz = ids16 != 0
                c0_v[j, sl] = jnp.where(nz, cnt16 & 0xFFFF, 0).astype(jnp.float32)
                c1_v[j, sl] = jnp.where(nz, cnt16 >> 16, 0).astype(jnp.float32)
        pltpu.sync_copy(c0_v, c0_hbm.at[row])
        pltpu.sync_copy(c1_v, c1_hbm.at[row])
        return carry

    lax.fori_loop(0, ROWS_PER_W, row_step, 0)


@functools.partial(
    pl.kernel,
    out_type=(
        jax.ShapeDtypeStruct((B, 4, 128), jnp.float32),
        jax.ShapeDtypeStruct((B, 4, 128), jnp.float32),
    ),
    mesh=_mesh,
    scratch_types=[
        pltpu.VMEM_SHARED((NS * HSIZE,), jnp.int32),
        pltpu.VMEM((4, 128), jnp.int32),
        pltpu.VMEM((4, 128), jnp.int32),
        pltpu.VMEM((4, 128), jnp.int32),
        pltpu.VMEM((4, 128), jnp.int32),
        pltpu.VMEM((4, 128), jnp.int32),
        pltpu.VMEM((4, 128), jnp.float32),
        pltpu.VMEM((4, 128), jnp.float32),
    ],
)
def _sc_count(*args):
    _sc_count_body(*args)


def _tc_encode_body(c0_ref, c1_ref, w1_ref, b1_ref, w2_ref, b2_ref,
                    src_out, dst_out):
    c0 = c0_ref[:, :L2]
    c1 = c1_ref[:, :L2]
    w1 = w1_ref[0, :]
    b1 = b1_ref[...]
    h = (jnp.maximum(c0[:, :, None] * w1[None, None, :] + b1[None], 0.0)
         + jnp.maximum(c1[:, :, None] * w1[None, None, :] + b1[None], 0.0))
    feat = jnp.dot(h.reshape(RB * L2, D), w2_ref[...],
                   preferred_element_type=jnp.float32)
    feat = feat + 2.0 * b2_ref[...]
    feat = feat.reshape(RB, L2, D)
    src_out[...] = feat[:, :SL, :]
    dst_out[...] = feat[:, SL:, :]


def _tc_encode(c0, c1, W1, b1, W2, b2):
    return pl.pallas_call(
        _tc_encode_body,
        grid=(B // RB,),
        in_specs=[
            pl.BlockSpec((RB, LP), lambda i: (i, 0)),
            pl.BlockSpec((RB, LP), lambda i: (i, 0)),
            pl.BlockSpec((1, D), lambda i: (0, 0)),
            pl.BlockSpec((1, D), lambda i: (0, 0)),
            pl.BlockSpec((D, D), lambda i: (0, 0)),
            pl.BlockSpec((1, D), lambda i: (0, 0)),
        ],
        out_specs=[
            pl.BlockSpec((RB, SL, D), lambda i: (i, 0, 0)),
            pl.BlockSpec((RB, DL, D), lambda i: (i, 0, 0)),
        ],
        out_shape=[
            jax.ShapeDtypeStruct((B, SL, D), jnp.float32),
            jax.ShapeDtypeStruct((B, DL, D), jnp.float32),
        ],
    )(c0, c1, W1, b1.reshape(1, D), W2, b2.reshape(1, D))


@jax.jit
def kernel(src_ids, dst_ids, W1, b1, W2, b2):
    ids = jnp.concatenate([src_ids.astype(jnp.int32),
                           dst_ids.astype(jnp.int32)], axis=1)
    ids_pad = jnp.pad(ids, ((0, 0), (0, LP - L2))).reshape(B, 4, 128)
    wvec = jnp.concatenate([
        jnp.full((SL,), 1, jnp.int32),
        jnp.full((DL,), 65536, jnp.int32),
        jnp.zeros((LP - L2,), jnp.int32),
    ]).reshape(4, 128)
    zvec = jnp.zeros((4, 128), jnp.int32)
    c0, c1 = _sc_count(ids_pad, wvec, zvec)
    src_feat, dst_feat = _tc_encode(c0.reshape(B, LP), c1.reshape(B, LP),
                                    W1, b1, W2, b2)
    return src_feat, dst_feat
